# baseline (speedup 1.0000x reference)
import functools
import math

import jax
import jax.numpy as jnp
from jax import lax
from jax.experimental import pallas as pl
from jax.experimental.pallas import tpu as pltpu
from jax.experimental.pallas import tpu_sc as plsc

M = 500000
PD = 128
SD = 128
H = 768
S = 2048
TOP_K = 5
THRESH = 0.5

NC = 2
NS = 16
NW = NC * NS
LANES = 16

SC_PT = 7680
M_SC = SC_PT * NW
M_TC = M - M_SC
CR = 256
NCHK = SC_PT // CR

NT = NS
_CHUNK = 20000
_NCHUNK = (M_TC + _CHUNK - 1) // _CHUNK
M_TC_PAD = _NCHUNK * _CHUNK
M2_PAD = ((M_TC_PAD + NT * LANES - 1) // (NT * LANES)) * NT * LANES
PER_T = M2_PAD // NT
NVREG = PER_T // LANES

NEG = -1e30
INV_SQRT_SD = 1.0 / math.sqrt(SD)


def _query_body(x_ref, wp_ref, bp_ref, out_ref):
    pc = jnp.mean(x_ref[...], axis=0)[None, :]
    q = jnp.dot(pc, wp_ref[...], preferred_element_type=jnp.float32)
    out_ref[...] = q + bp_ref[...]


def _project_query(x2d, Wp, bp2d):
    return pl.pallas_call(
        _query_body,
        out_shape=jax.ShapeDtypeStruct((1, PD), jnp.float32),
    )(x2d, Wp, bp2d)


def _scores_body(q_ref, pm_ref, conf_ref, use_ref, succ_ref, out_ref):
    sim = lax.dot_general(q_ref[...], pm_ref[...],
                          dimension_numbers=(((1,), (1,)), ((), ())),
                          preferred_element_type=jnp.float32)[0]
    use = use_ref[0, 0, :]
    scores = ((sim + 0.1 * jnp.log(use + 1.0))
              + 0.2 * conf_ref[0, 0, :]
              + 0.3 * (succ_ref[0, 0, :] / (use + 1e-8)))
    row = (pl.program_id(0) * _CHUNK
           + lax.broadcasted_iota(jnp.int32, (_CHUNK,), 0))
    out_ref[0, 0, :] = jnp.where(row < M_TC, scores, NEG)


def _compute_scores(query, problem_memory, confidence_memory, usage, success):
    vec3 = lambda v: v[:M_TC_PAD].reshape(_NCHUNK, 1, _CHUNK)
    out = pl.pallas_call(
        _scores_body,
        grid=(_NCHUNK,),
        in_specs=[
            pl.BlockSpec((1, PD), lambda i: (0, 0)),
            pl.BlockSpec((_CHUNK, PD), lambda i: (i, 0)),
            pl.BlockSpec((1, 1, _CHUNK), lambda i: (i, 0, 0)),
            pl.BlockSpec((1, 1, _CHUNK), lambda i: (i, 0, 0)),
            pl.BlockSpec((1, 1, _CHUNK), lambda i: (i, 0, 0)),
        ],
        out_specs=pl.BlockSpec((1, 1, _CHUNK), lambda i: (i, 0, 0)),
        out_shape=jax.ShapeDtypeStruct((_NCHUNK, 1, _CHUNK), jnp.float32),
    )(query, problem_memory, vec3(confidence_memory[:, 0]), vec3(usage),
      vec3(success))
    return out.reshape(M_TC_PAD)


def _boost_body(conf_ref, use_ref, succ_ref, out_ref):
    use = use_ref[...]
    out_ref[...] = ((0.1 * jnp.log(use + 1.0))
                    + 0.2 * conf_ref[...]
                    + 0.3 * (succ_ref[...] / (use + 1e-8)))


def _compute_boost(conf_sc, use_sc, succ_sc):
    return pl.pallas_call(
        _boost_body,
        out_shape=jax.ShapeDtypeStruct((M_SC,), jnp.float32),
    )(conf_sc, use_sc, succ_sc)


def _lane_iota():
    return lax.iota(jnp.int32, LANES)


def _extract_at_lane(vec, lane):
    return jnp.sum(jnp.where(_lane_iota() == lane, vec, 0))


def _merge_rows(rows_s, rows_i):
    lanes = _lane_iota()
    nrows = len(rows_s)
    top_s = jnp.full((LANES,), NEG, jnp.float32)
    top_i = jnp.zeros((LANES,), jnp.int32)
    chosen = []
    for k in range(TOP_K):
        m = jnp.full((LANES,), NEG, jnp.float32)
        arow = jnp.zeros((LANES,), jnp.int32)
        for r in range(nrows):
            v = rows_s[r]
            flatpos = r * LANES + lanes
            for c in chosen:
                v = jnp.where(flatpos == c, NEG, v)
            pred = v > m
            m = jnp.where(pred, v, m)
            arow = jnp.where(pred, r, arow)
        gm = jnp.max(m)
        pred = (m == gm)
        first = jnp.logical_and(pred, jnp.cumsum(pred.astype(jnp.int32)) == 1)
        lane = jnp.sum(jnp.where(first, lanes, 0))
        rbest = jnp.sum(jnp.where(first, arow, 0))
        chosen.append(rbest * LANES + lane)
        acc = jnp.zeros((LANES,), jnp.int32)
        for r in range(nrows):
            acc = jnp.where(jnp.logical_and(rbest == r, lanes == lane),
                            rows_i[r], acc)
        gi = jnp.sum(acc)
        top_s = jnp.where(lanes == k, gm, top_s)
        top_i = jnp.where(lanes == k, gi, top_i)
    return top_s, top_i


def _local_top5(score_ref, base):
    lanes = _lane_iota()

    def body(j, carry):
        ts0, ts1, ts2, ts3, ts4, ti0, ti1, ti2, ti3, ti4 = carry
        cv = score_ref[pl.ds(j * LANES, LANES)]
        ci = (base + j * LANES) + lanes
        ts = [ts0, ts1, ts2, ts3, ts4]
        ti = [ti0, ti1, ti2, ti3, ti4]
        for k in range(TOP_K):
            pred = cv > ts[k]
            ns = jnp.where(pred, cv, ts[k])
            ni = jnp.where(pred, ci, ti[k])
            cv = jnp.where(pred, ts[k], cv)
            ci = jnp.where(pred, ti[k], ci)
            ts[k], ti[k] = ns, ni
        return (*ts, *ti)

    init = ([jnp.full((LANES,), NEG, jnp.float32)] * TOP_K
            + [jnp.zeros((LANES,), jnp.int32)] * TOP_K)
    carry = lax.fori_loop(0, NVREG, body, tuple(init))
    return _merge_rows(list(carry[:TOP_K]), list(carry[TOP_K:]))


def _sc_matvec_body(q_hbm, pm_hbm, boost_hbm, cs_out, ci_out,
                    q_v, boost_v, buf_v, trans_v, stage_s, stage_i,
                    sem0, sem1):
    wid = lax.axis_index("s") * NC + lax.axis_index("c")
    row0 = M_TC + wid * SC_PT
    lanes = _lane_iota()

    pltpu.sync_copy(q_hbm.at[0], q_v)
    pltpu.sync_copy(boost_hbm.at[pl.ds(wid * SC_PT, SC_PT)], boost_v)
    qv = [q_v[pl.ds(c * LANES, LANES)] for c in range(PD // LANES)]
    sems = (sem0, sem1)

    pltpu.async_copy(pm_hbm.at[pl.ds(row0, CR)], buf_v.at[0], sem0)

    def chunk(parity, cidx, carry):
        def group(g, carry):
            ts0, ts1, ts2, ts3, ts4, ti0, ti1, ti2, ti3, ti4 = carry
            for r in range(LANES):
                acc = buf_v[parity, g * LANES + r,
                            pl.ds(0, LANES)] * qv[0]
                for c in range(1, PD // LANES):
                    acc = acc + (buf_v[parity, g * LANES + r,
                                       pl.ds(c * LANES, LANES)] * qv[c])
                trans_v[r] = acc
            cv = boost_v[pl.ds(cidx * CR + g * LANES, LANES)]
            for c in range(LANES):
                cv = cv + plsc.load_gather(
                    trans_v, [lanes, jnp.full((LANES,), c, jnp.int32)])
            ci = (row0 + cidx * CR + g * LANES) + lanes
            ts = [ts0, ts1, ts2, ts3, ts4]
            ti = [ti0, ti1, ti2, ti3, ti4]
            for k in range(TOP_K):
                pred = cv > ts[k]
                ns = jnp.where(pred, cv, ts[k])
                ni = jnp.where(pred, ci, ti[k])
                cv = jnp.where(pred, ts[k], cv)
                ci = jnp.where(pred, ti[k], ci)
                ts[k], ti[k] = ns, ni
            return (*ts, *ti)

        return lax.fori_loop(0, CR // LANES, group, carry)

    def pair(i, carry):
        for b in range(2):
            cidx = 2 * i + b
            pltpu.make_async_copy(
                pm_hbm.at[pl.ds(row0, CR)], buf_v.at[b], sems[b]).wait()
            nxt = cidx + 1

            @pl.when(nxt < NCHK)
            def _():
                pltpu.async_copy(
                    pm_hbm.at[pl.ds(row0 + nxt * CR, CR)],
                    buf_v.at[1 - b], sems[1 - b])

            carry = chunk(b, cidx, carry)
        return carry

    init = ([jnp.full((LANES,), NEG, jnp.float32)] * TOP_K
            + [jnp.zeros((LANES,), jnp.int32)] * TOP_K)
    carry = lax.fori_loop(0, NCHK // 2, pair, tuple(init))

    loc_s, loc_i = _merge_rows(list(carry[:TOP_K]), list(carry[TOP_K:]))
    stage_s[...] = loc_s
    stage_i[...] = loc_i
    pltpu.sync_copy(stage_s, cs_out.at[wid])
    pltpu.sync_copy(stage_i, ci_out.at[wid])


def _sc_matvec(query, problem_memory, boost_sc):
    mesh = plsc.VectorSubcoreMesh(
        core_axis_name="c", subcore_axis_name="s",
        num_cores=NC, num_subcores=NS)
    fn = pl.kernel(
        _sc_matvec_body,
        out_type=(
            jax.ShapeDtypeStruct((NW, LANES), jnp.float32),
            jax.ShapeDtypeStruct((NW, LANES), jnp.int32),
        ),
        mesh=mesh,
        scratch_types=[
            pltpu.VMEM((PD,), jnp.float32),
            pltpu.VMEM((SC_PT,), jnp.float32),
            pltpu.VMEM((2, CR, PD), jnp.float32),
            pltpu.VMEM((LANES, LANES), jnp.float32),
            pltpu.VMEM((LANES,), jnp.float32),
            pltpu.VMEM((LANES,), jnp.int32),
            pltpu.SemaphoreType.DMA,
            pltpu.SemaphoreType.DMA,
        ],
        compiler_params=pltpu.CompilerParams(needs_layout_passes=False),
    )
    return fn(query, problem_memory, boost_sc)


def _sc_body(scores_hbm, sol_hbm, sc_cs, sc_ci,
             ts_out, ti_out, w_out, comb_out, misc_out, hbm_s, hbm_i,
             local_v, cand_s_v, cand_i_v, cand2_s_v, cand2_i_v,
             stage_s, stage_i, idx_v, rows_v, comb_v, sem):
    cid = lax.axis_index("c")
    sid = lax.axis_index("s")
    base = sid * PER_T
    lanes = _lane_iota()

    @pl.when(cid == 0)
    def _():
        pltpu.sync_copy(scores_hbm.at[pl.ds(base, PER_T)], local_v)

        loc_s, loc_i = _local_top5(local_v, base)

        stage_s[...] = loc_s
        stage_i[...] = loc_i
        pltpu.sync_copy(stage_s, hbm_s.at[sid])
        pltpu.sync_copy(stage_i, hbm_i.at[sid])

    plsc.subcore_barrier()

    @pl.when(jnp.logical_and(cid == 0, sid == 0))
    def _():
        pltpu.sync_copy(hbm_s, cand_s_v)
        pltpu.sync_copy(hbm_i, cand_i_v)
        pltpu.sync_copy(sc_cs, cand2_s_v)
        pltpu.sync_copy(sc_ci, cand2_i_v)

        top_s, top_i = _merge_rows(
            [cand_s_v[r] for r in range(NT)]
            + [cand2_s_v[r] for r in range(NW)],
            [cand_i_v[r] for r in range(NT)]
            + [cand2_i_v[r] for r in range(NW)])

        valid = lanes < TOP_K
        s0 = jnp.max(top_s)
        logits = (top_s - s0) * INV_SQRT_SD
        w_un = jnp.where(valid, jnp.exp(logits), 0.0)
        weights = w_un / jnp.sum(w_un)

        idx_v[...] = jnp.where(valid, top_i, 0)
        pltpu.async_copy(sol_hbm.at[idx_v], rows_v, sem).wait()

        for c in range(SD // LANES):
            sl = pl.ds(c * LANES, LANES)
            acc = jnp.zeros((LANES,), jnp.float32)
            for k in range(TOP_K):
                wk = _extract_at_lane(weights, k)
                acc = acc + wk * rows_v[k, sl]
            comb_v[sl] = acc

        conf = s0
        n_used = jnp.sum(jnp.where(
            jnp.logical_and(valid, top_s > THRESH), 1.0, 0.0))
        misc = jnp.where(lanes == 0, conf, 0.0)
        misc = jnp.where(lanes == 1, n_used, misc)

        stage_s[...] = jnp.where(valid, top_s, 0.0)
        pltpu.sync_copy(stage_s, ts_out)
        stage_i[...] = top_i
        pltpu.sync_copy(stage_i, ti_out)
        stage_s[...] = weights
        pltpu.sync_copy(stage_s, w_out)
        pltpu.sync_copy(comb_v, comb_out)
        stage_s[...] = misc
        pltpu.sync_copy(stage_s, misc_out)


def _sc_topk(scores_pad, solution_memory, sc_cs, sc_ci):
    mesh = plsc.VectorSubcoreMesh(
        core_axis_name="c", subcore_axis_name="s",
        num_cores=NC, num_subcores=NS)
    fn = pl.kernel(
        _sc_body,
        out_type=(
            jax.ShapeDtypeStruct((LANES,), jnp.float32),
            jax.ShapeDtypeStruct((LANES,), jnp.int32),
            jax.ShapeDtypeStruct((LANES,), jnp.float32),
            jax.ShapeDtypeStruct((SD,), jnp.float32),
            jax.ShapeDtypeStruct((LANES,), jnp.float32),
            jax.ShapeDtypeStruct((NT, LANES), jnp.float32),
            jax.ShapeDtypeStruct((NT, LANES), jnp.int32),
        ),
        mesh=mesh,
        scratch_types=[
            pltpu.VMEM((PER_T,), jnp.float32),
            pltpu.VMEM((NT, LANES), jnp.float32),
            pltpu.VMEM((NT, LANES), jnp.int32),
            pltpu.VMEM((NW, LANES), jnp.float32),
            pltpu.VMEM((NW, LANES), jnp.int32),
            pltpu.VMEM((LANES,), jnp.float32),
            pltpu.VMEM((LANES,), jnp.int32),
            pltpu.VMEM((LANES,), jnp.int32),
            pltpu.VMEM((LANES, SD), jnp.float32),
            pltpu.VMEM((SD,), jnp.float32),
            pltpu.SemaphoreType.DMA,
        ],
        compiler_params=pltpu.CompilerParams(needs_layout_passes=False),
    )
    return fn(scores_pad, solution_memory, sc_cs, sc_ci)


def _output_body(x_ref, comb_ref, wo_ref, bo_ref, out_ref):
    e = jnp.dot(comb_ref[0, :], wo_ref[...],
                preferred_element_type=jnp.float32,
                precision=lax.Precision.HIGHEST) + bo_ref[0, :]
    g = jnp.dot(x_ref[...], e, preferred_element_type=jnp.float32,
                precision=lax.Precision.HIGHEST)
    gate = jax.nn.sigmoid(g)[:, None]
    out_ref[...] = gate * e[None, :] + (1.0 - gate) * x_ref[...]


def _blend_output(x2d, comb2d, Wo, bo2d):
    return pl.pallas_call(
        _output_body,
        out_shape=jax.ShapeDtypeStruct((S, H), jnp.float32),
    )(x2d, comb2d, Wo, bo2d)


@jax.jit
def kernel(x, problem_memory, solution_memory, confidence_memory,
           Wp, bp, Wo, bo, pattern_usage, pattern_success):
    B, S_, H_ = x.shape
    x2d = x.reshape(S_, H_)

    query = _project_query(x2d, Wp, bp.reshape(1, PD))
    scores = _compute_scores(query, problem_memory, confidence_memory,
                             pattern_usage, pattern_success)
    scores_pad = jnp.concatenate(
        [scores, jnp.full((M2_PAD - M_TC_PAD,), NEG, jnp.float32)])
    boost_sc = _compute_boost(confidence_memory[M_TC:, 0],
                              pattern_usage[M_TC:], pattern_success[M_TC:])
    sc_cs, sc_ci = _sc_matvec(query, problem_memory, boost_sc)

    ts, ti, w, comb, misc, _, _ = _sc_topk(scores_pad, solution_memory,
                                           sc_cs, sc_ci)

    out2d = _blend_output(x2d, comb.reshape(1, SD), Wo, bo.reshape(1, H))

    output = out2d.reshape(B, S_, H_)
    top_scores = ts[:TOP_K][None, :]
    top_indices = ti[:TOP_K][None, :]
    attention_weights = w[:TOP_K][None, :]
    confidence = misc[0:1]
    num_patterns_used = misc[1:2].astype(jnp.int32)
    return (output, top_indices, top_scores, attention_weights,
            confidence, num_patterns_used)


# speedup vs baseline: 1.0005x; 1.0005x over previous
import functools
import math

import jax
import jax.numpy as jnp
from jax import lax
from jax.experimental import pallas as pl
from jax.experimental.pallas import tpu as pltpu
from jax.experimental.pallas import tpu_sc as plsc

M = 500000
PD = 128
SD = 128
H = 768
S = 2048
TOP_K = 5
THRESH = 0.5

NC = 2
NS = 16
NW = NC * NS
LANES = 16

SC_PT = 7680
M_SC = SC_PT * NW
M_TC = M - M_SC
CR = 256
NCHK = SC_PT // CR

NT = NS
_CHUNK = 20000
_NCHUNK = (M_TC + _CHUNK - 1) // _CHUNK
M_TC_PAD = _NCHUNK * _CHUNK
M2_PAD = ((M_TC_PAD + NT * LANES - 1) // (NT * LANES)) * NT * LANES
PER_T = M2_PAD // NT
NVREG = PER_T // LANES

NEG = -1e30
INV_SQRT_SD = 1.0 / math.sqrt(SD)


def _query_body(x_ref, wp_ref, bp_ref, out_ref):
    pc = jnp.mean(x_ref[...], axis=0)[None, :]
    q = jnp.dot(pc, wp_ref[...], preferred_element_type=jnp.float32)
    out_ref[...] = q + bp_ref[...]


def _project_query(x2d, Wp, bp2d):
    return pl.pallas_call(
        _query_body,
        out_shape=jax.ShapeDtypeStruct((1, PD), jnp.float32),
    )(x2d, Wp, bp2d)


def _scores_body(q_ref, pm_ref, conf_ref, use_ref, succ_ref, out_ref):
    sim = lax.dot_general(q_ref[...], pm_ref[...],
                          dimension_numbers=(((1,), (1,)), ((), ())),
                          preferred_element_type=jnp.float32)[0]
    use = use_ref[0, 0, :]
    scores = ((sim + 0.1 * jnp.log(use + 1.0))
              + 0.2 * conf_ref[0, 0, :]
              + 0.3 * (succ_ref[0, 0, :] / (use + 1e-8)))
    row = (pl.program_id(0) * _CHUNK
           + lax.broadcasted_iota(jnp.int32, (_CHUNK,), 0))
    out_ref[0, 0, :] = jnp.where(row < M_TC, scores, NEG)


def _compute_scores(query, problem_memory, confidence_memory, usage, success):
    vec3 = lambda v: v[:M_TC_PAD].reshape(_NCHUNK, 1, _CHUNK)
    out = pl.pallas_call(
        _scores_body,
        grid=(_NCHUNK,),
        in_specs=[
            pl.BlockSpec((1, PD), lambda i: (0, 0)),
            pl.BlockSpec((_CHUNK, PD), lambda i: (i, 0)),
            pl.BlockSpec((1, 1, _CHUNK), lambda i: (i, 0, 0)),
            pl.BlockSpec((1, 1, _CHUNK), lambda i: (i, 0, 0)),
            pl.BlockSpec((1, 1, _CHUNK), lambda i: (i, 0, 0)),
        ],
        out_specs=pl.BlockSpec((1, 1, _CHUNK), lambda i: (i, 0, 0)),
        out_shape=jax.ShapeDtypeStruct((_NCHUNK, 1, _CHUNK), jnp.float32),
    )(query, problem_memory, vec3(confidence_memory[:, 0]), vec3(usage),
      vec3(success))
    return out.reshape(M_TC_PAD)


def _boost_body(conf_ref, use_ref, succ_ref, out_ref):
    use = use_ref[...]
    out_ref[...] = ((0.1 * jnp.log(use + 1.0))
                    + 0.2 * conf_ref[...]
                    + 0.3 * (succ_ref[...] / (use + 1e-8)))


def _compute_boost(conf_sc, use_sc, succ_sc):
    return pl.pallas_call(
        _boost_body,
        out_shape=jax.ShapeDtypeStruct((M_SC,), jnp.float32),
    )(conf_sc, use_sc, succ_sc)


def _lane_iota():
    return lax.iota(jnp.int32, LANES)


def _extract_at_lane(vec, lane):
    return jnp.sum(jnp.where(_lane_iota() == lane, vec, 0))


def _merge_rows(rows_s, rows_i):
    lanes = _lane_iota()
    nrows = len(rows_s)
    top_s = jnp.full((LANES,), NEG, jnp.float32)
    top_i = jnp.zeros((LANES,), jnp.int32)
    chosen = []
    for k in range(TOP_K):
        m = jnp.full((LANES,), NEG, jnp.float32)
        arow = jnp.zeros((LANES,), jnp.int32)
        for r in range(nrows):
            v = rows_s[r]
            flatpos = r * LANES + lanes
            for c in chosen:
                v = jnp.where(flatpos == c, NEG, v)
            pred = v > m
            m = jnp.where(pred, v, m)
            arow = jnp.where(pred, r, arow)
        gm = jnp.max(m)
        pred = (m == gm)
        first = jnp.logical_and(pred, jnp.cumsum(pred.astype(jnp.int32)) == 1)
        lane = jnp.sum(jnp.where(first, lanes, 0))
        rbest = jnp.sum(jnp.where(first, arow, 0))
        chosen.append(rbest * LANES + lane)
        acc = jnp.zeros((LANES,), jnp.int32)
        for r in range(nrows):
            acc = jnp.where(jnp.logical_and(rbest == r, lanes == lane),
                            rows_i[r], acc)
        gi = jnp.sum(acc)
        top_s = jnp.where(lanes == k, gm, top_s)
        top_i = jnp.where(lanes == k, gi, top_i)
    return top_s, top_i


def _local_top5(score_ref, base):
    lanes = _lane_iota()

    def body(j, carry):
        ts0, ts1, ts2, ts3, ts4, ti0, ti1, ti2, ti3, ti4 = carry
        cv = score_ref[pl.ds(j * LANES, LANES)]
        ci = (base + j * LANES) + lanes
        ts = [ts0, ts1, ts2, ts3, ts4]
        ti = [ti0, ti1, ti2, ti3, ti4]
        for k in range(TOP_K):
            pred = cv > ts[k]
            ns = jnp.where(pred, cv, ts[k])
            ni = jnp.where(pred, ci, ti[k])
            cv = jnp.where(pred, ts[k], cv)
            ci = jnp.where(pred, ti[k], ci)
            ts[k], ti[k] = ns, ni
        return (*ts, *ti)

    init = ([jnp.full((LANES,), NEG, jnp.float32)] * TOP_K
            + [jnp.zeros((LANES,), jnp.int32)] * TOP_K)
    carry = lax.fori_loop(0, NVREG, body, tuple(init))
    return _merge_rows(list(carry[:TOP_K]), list(carry[TOP_K:]))


def _sc_matvec_body(q_hbm, pm_hbm, boost_hbm, cs_out, ci_out,
                    q_v, boost_v, buf_v, trans_v, stage_s, stage_i,
                    sem0, sem1):
    wid = lax.axis_index("s") * NC + lax.axis_index("c")
    row0 = M_TC + wid * SC_PT
    lanes = _lane_iota()

    pltpu.sync_copy(q_hbm.at[0], q_v)
    pltpu.sync_copy(boost_hbm.at[pl.ds(wid * SC_PT, SC_PT)], boost_v)
    qv = [q_v[pl.ds(c * LANES, LANES)] for c in range(PD // LANES)]
    sems = (sem0, sem1)

    pltpu.async_copy(pm_hbm.at[pl.ds(row0, CR)], buf_v.at[0], sem0)

    def chunk(parity, cidx, carry):
        def group(g, carry):
            ts0, ts1, ts2, ts3, ts4, ti0, ti1, ti2, ti3, ti4 = carry
            for r in range(LANES):
                acc = buf_v[parity, g * LANES + r,
                            pl.ds(0, LANES)] * qv[0]
                for c in range(1, PD // LANES):
                    acc = acc + (buf_v[parity, g * LANES + r,
                                       pl.ds(c * LANES, LANES)] * qv[c])
                trans_v[r] = acc
            cv = boost_v[pl.ds(cidx * CR + g * LANES, LANES)]
            for c in range(LANES):
                cv = cv + plsc.load_gather(
                    trans_v, [lanes, jnp.full((LANES,), c, jnp.int32)])
            ci = (row0 + cidx * CR + g * LANES) + lanes
            ts = [ts0, ts1, ts2, ts3, ts4]
            ti = [ti0, ti1, ti2, ti3, ti4]
            for k in range(TOP_K):
                pred = cv > ts[k]
                ns = jnp.where(pred, cv, ts[k])
                ni = jnp.where(pred, ci, ti[k])
                cv = jnp.where(pred, ts[k], cv)
                ci = jnp.where(pred, ti[k], ci)
                ts[k], ti[k] = ns, ni
            return (*ts, *ti)

        return lax.fori_loop(0, CR // LANES, group, carry)

    def pair(i, carry):
        for b in range(2):
            cidx = 2 * i + b
            pltpu.make_async_copy(
                pm_hbm.at[pl.ds(row0, CR)], buf_v.at[b], sems[b]).wait()
            nxt = cidx + 1

            @pl.when(nxt < NCHK)
            def _():
                pltpu.async_copy(
                    pm_hbm.at[pl.ds(row0 + nxt * CR, CR)],
                    buf_v.at[1 - b], sems[1 - b])

            carry = chunk(b, cidx, carry)
        return carry

    init = ([jnp.full((LANES,), NEG, jnp.float32)] * TOP_K
            + [jnp.zeros((LANES,), jnp.int32)] * TOP_K)
    carry = lax.fori_loop(0, NCHK // 2, pair, tuple(init))

    loc_s, loc_i = _merge_rows(list(carry[:TOP_K]), list(carry[TOP_K:]))
    stage_s[...] = loc_s
    stage_i[...] = loc_i
    pltpu.sync_copy(stage_s, cs_out.at[wid])
    pltpu.sync_copy(stage_i, ci_out.at[wid])


def _sc_matvec(query, problem_memory, boost_sc):
    mesh = plsc.VectorSubcoreMesh(
        core_axis_name="c", subcore_axis_name="s",
        num_cores=NC, num_subcores=NS)
    fn = pl.kernel(
        _sc_matvec_body,
        out_type=(
            jax.ShapeDtypeStruct((NW, LANES), jnp.float32),
            jax.ShapeDtypeStruct((NW, LANES), jnp.int32),
        ),
        mesh=mesh,
        scratch_types=[
            pltpu.VMEM((PD,), jnp.float32),
            pltpu.VMEM((SC_PT,), jnp.float32),
            pltpu.VMEM((2, CR, PD), jnp.float32),
            pltpu.VMEM((LANES, LANES), jnp.float32),
            pltpu.VMEM((LANES,), jnp.float32),
            pltpu.VMEM((LANES,), jnp.int32),
            pltpu.SemaphoreType.DMA,
            pltpu.SemaphoreType.DMA,
        ],
        compiler_params=pltpu.CompilerParams(needs_layout_passes=False),
    )
    return fn(query, problem_memory, boost_sc)


def _sc_body(scores_hbm, sol_hbm, sc_cs, sc_ci,
             ts_out, ti_out, w_out, comb_out, misc_out, hbm_s, hbm_i,
             local_v, cand_s_v, cand_i_v, cand2_s_v, cand2_i_v,
             stage_s, stage_i, idx_v, rows_v, comb_v, sem):
    cid = lax.axis_index("c")
    sid = lax.axis_index("s")
    base = sid * PER_T
    lanes = _lane_iota()

    @pl.when(cid == 0)
    def _():
        pltpu.sync_copy(scores_hbm.at[pl.ds(base, PER_T)], local_v)

        loc_s, loc_i = _local_top5(local_v, base)

        stage_s[...] = loc_s
        stage_i[...] = loc_i
        pltpu.sync_copy(stage_s, hbm_s.at[sid])
        pltpu.sync_copy(stage_i, hbm_i.at[sid])

    plsc.subcore_barrier()

    @pl.when(jnp.logical_and(cid == 0, sid == 0))
    def _():
        pltpu.sync_copy(hbm_s, cand_s_v)
        pltpu.sync_copy(hbm_i, cand_i_v)
        pltpu.sync_copy(sc_cs, cand2_s_v)
        pltpu.sync_copy(sc_ci, cand2_i_v)

        top_s, top_i = _merge_rows(
            [cand_s_v[r] for r in range(NT)]
            + [cand2_s_v[r] for r in range(NW)],
            [cand_i_v[r] for r in range(NT)]
            + [cand2_i_v[r] for r in range(NW)])

        valid = lanes < TOP_K
        s0 = jnp.max(top_s)
        logits = (top_s - s0) * INV_SQRT_SD
        w_un = jnp.where(valid, jnp.exp(logits), 0.0)
        weights = w_un / jnp.sum(w_un)

        idx_v[...] = jnp.where(valid, top_i, 0)
        pltpu.async_copy(sol_hbm.at[idx_v], rows_v, sem).wait()

        for c in range(SD // LANES):
            sl = pl.ds(c * LANES, LANES)
            acc = jnp.zeros((LANES,), jnp.float32)
            for k in range(TOP_K):
                wk = _extract_at_lane(weights, k)
                acc = acc + wk * rows_v[k, sl]
            comb_v[sl] = acc

        conf = s0
        n_used = jnp.sum(jnp.where(
            jnp.logical_and(valid, top_s > THRESH), 1.0, 0.0))
        misc = jnp.where(lanes == 0, conf, 0.0)
        misc = jnp.where(lanes == 1, n_used, misc)

        stage_s[...] = jnp.where(valid, top_s, 0.0)
        pltpu.sync_copy(stage_s, ts_out)
        stage_i[...] = top_i
        pltpu.sync_copy(stage_i, ti_out)
        stage_s[...] = weights
        pltpu.sync_copy(stage_s, w_out)
        pltpu.sync_copy(comb_v, comb_out)
        stage_s[...] = misc
        pltpu.sync_copy(stage_s, misc_out)


def _sc_topk(scores_pad, solution_memory, sc_cs, sc_ci):
    mesh = plsc.VectorSubcoreMesh(
        core_axis_name="c", subcore_axis_name="s",
        num_cores=NC, num_subcores=NS)
    fn = pl.kernel(
        _sc_body,
        out_type=(
            jax.ShapeDtypeStruct((LANES,), jnp.float32),
            jax.ShapeDtypeStruct((LANES,), jnp.int32),
            jax.ShapeDtypeStruct((LANES,), jnp.float32),
            jax.ShapeDtypeStruct((SD,), jnp.float32),
            jax.ShapeDtypeStruct((LANES,), jnp.float32),
            jax.ShapeDtypeStruct((NT, LANES), jnp.float32),
            jax.ShapeDtypeStruct((NT, LANES), jnp.int32),
        ),
        mesh=mesh,
        scratch_types=[
            pltpu.VMEM((PER_T,), jnp.float32),
            pltpu.VMEM((NT, LANES), jnp.float32),
            pltpu.VMEM((NT, LANES), jnp.int32),
            pltpu.VMEM((NW, LANES), jnp.float32),
            pltpu.VMEM((NW, LANES), jnp.int32),
            pltpu.VMEM((LANES,), jnp.float32),
            pltpu.VMEM((LANES,), jnp.int32),
            pltpu.VMEM((LANES,), jnp.int32),
            pltpu.VMEM((LANES, SD), jnp.float32),
            pltpu.VMEM((SD,), jnp.float32),
            pltpu.SemaphoreType.DMA,
        ],
        compiler_params=pltpu.CompilerParams(needs_layout_passes=False),
    )
    return fn(scores_pad, solution_memory, sc_cs, sc_ci)


def _output_body(x_ref, comb_ref, wo_ref, bo_ref, out_ref):
    e = jnp.dot(comb_ref[0, :], wo_ref[...],
                preferred_element_type=jnp.float32,
                precision=lax.Precision.HIGHEST) + bo_ref[0, :]
    g = jnp.dot(x_ref[...], e, preferred_element_type=jnp.float32,
                precision=lax.Precision.HIGHEST)
    gate = jax.nn.sigmoid(g)[:, None]
    out_ref[...] = gate * e[None, :] + (1.0 - gate) * x_ref[...]


def _blend_output(x2d, comb2d, Wo, bo2d):
    return pl.pallas_call(
        _output_body,
        out_shape=jax.ShapeDtypeStruct((S, H), jnp.float32),
    )(x2d, comb2d, Wo, bo2d)


@jax.jit
def kernel(x, problem_memory, solution_memory, confidence_memory,
           Wp, bp, Wo, bo, pattern_usage, pattern_success):
    B, S_, H_ = x.shape
    x2d = x.reshape(S_, H_)

    query = _project_query(x2d, Wp, bp.reshape(1, PD))
    boost_sc = _compute_boost(confidence_memory[M_TC:, 0],
                              pattern_usage[M_TC:], pattern_success[M_TC:])
    sc_cs, sc_ci = _sc_matvec(query, problem_memory, boost_sc)
    scores = _compute_scores(query, problem_memory, confidence_memory,
                             pattern_usage, pattern_success)
    scores_pad = jnp.concatenate(
        [scores, jnp.full((M2_PAD - M_TC_PAD,), NEG, jnp.float32)])

    ts, ti, w, comb, misc, _, _ = _sc_topk(scores_pad, solution_memory,
                                           sc_cs, sc_ci)

    out2d = _blend_output(x2d, comb.reshape(1, SD), Wo, bo.reshape(1, H))

    output = out2d.reshape(B, S_, H_)
    top_scores = ts[:TOP_K][None, :]
    top_indices = ti[:TOP_K][None, :]
    attention_weights = w[:TOP_K][None, :]
    confidence = misc[0:1]
    num_patterns_used = misc[1:2].astype(jnp.int32)
    return (output, top_indices, top_scores, attention_weights,
            confidence, num_patterns_used)


# speedup vs baseline: 1.3231x; 1.3224x over previous
import functools
import math

import jax
import jax.numpy as jnp
from jax import lax
from jax.experimental import pallas as pl
from jax.experimental.pallas import tpu as pltpu
from jax.experimental.pallas import tpu_sc as plsc

M = 500000
PD = 128
SD = 128
H = 768
S = 2048
TOP_K = 5
THRESH = 0.5

NC = 2
NS = 16
NW = NC * NS
LANES = 16

NT = NS
PER_T = ((M + NT * LANES - 1) // (NT * LANES)) * LANES
M_PAD = PER_T * NT
NVREG = PER_T // LANES

NEG = -1e30
INV_SQRT_SD = 1.0 / math.sqrt(SD)


def _query_body(x_ref, wp_ref, bp_ref, out_ref):
    pc = jnp.mean(x_ref[...], axis=0)[None, :]
    q = jnp.dot(pc, wp_ref[...], preferred_element_type=jnp.float32)
    out_ref[...] = q + bp_ref[...]


def _project_query(x2d, Wp, bp2d):
    return pl.pallas_call(
        _query_body,
        out_shape=jax.ShapeDtypeStruct((1, PD), jnp.float32),
    )(x2d, Wp, bp2d)


_CHUNK = 25000
_NCHUNK = M // _CHUNK


def _scores_body(q_ref, pm_ref, conf_ref, use_ref, succ_ref, out_ref):
    sim = lax.dot_general(q_ref[...], pm_ref[...],
                          dimension_numbers=(((1,), (1,)), ((), ())),
                          preferred_element_type=jnp.float32)[0]
    use = use_ref[0, 0, :]
    out_ref[0, 0, :] = ((sim + 0.1 * jnp.log(use + 1.0))
                        + 0.2 * conf_ref[0, 0, :]
                        + 0.3 * (succ_ref[0, 0, :] / (use + 1e-8)))


def _compute_scores(query, problem_memory, confidence_memory, usage, success):
    vec3 = lambda v: v.reshape(_NCHUNK, 1, _CHUNK)
    out = pl.pallas_call(
        _scores_body,
        grid=(_NCHUNK,),
        in_specs=[
            pl.BlockSpec((1, PD), lambda i: (0, 0)),
            pl.BlockSpec((_CHUNK, PD), lambda i: (i, 0)),
            pl.BlockSpec((1, 1, _CHUNK), lambda i: (i, 0, 0)),
            pl.BlockSpec((1, 1, _CHUNK), lambda i: (i, 0, 0)),
            pl.BlockSpec((1, 1, _CHUNK), lambda i: (i, 0, 0)),
        ],
        out_specs=pl.BlockSpec((1, 1, _CHUNK), lambda i: (i, 0, 0)),
        out_shape=jax.ShapeDtypeStruct((_NCHUNK, 1, _CHUNK), jnp.float32),
    )(query, problem_memory, vec3(confidence_memory[:, 0]), vec3(usage),
      vec3(success))
    return out.reshape(M)


def _lane_iota():
    return lax.iota(jnp.int32, LANES)


def _extract_at_lane(vec, lane):
    return jnp.sum(jnp.where(_lane_iota() == lane, vec, 0))


def _merge_rows(rows_s, rows_i):
    lanes = _lane_iota()
    nrows = len(rows_s)
    top_s = jnp.full((LANES,), NEG, jnp.float32)
    top_i = jnp.zeros((LANES,), jnp.int32)
    chosen = []
    for k in range(TOP_K):
        m = jnp.full((LANES,), NEG, jnp.float32)
        arow = jnp.zeros((LANES,), jnp.int32)
        for r in range(nrows):
            v = rows_s[r]
            flatpos = r * LANES + lanes
            for c in chosen:
                v = jnp.where(flatpos == c, NEG, v)
            pred = v > m
            m = jnp.where(pred, v, m)
            arow = jnp.where(pred, r, arow)
        gm = jnp.max(m)
        pred = (m == gm)
        first = jnp.logical_and(pred, jnp.cumsum(pred.astype(jnp.int32)) == 1)
        lane = jnp.sum(jnp.where(first, lanes, 0))
        rbest = jnp.sum(jnp.where(first, arow, 0))
        chosen.append(rbest * LANES + lane)
        acc = jnp.zeros((LANES,), jnp.int32)
        for r in range(nrows):
            acc = jnp.where(jnp.logical_and(rbest == r, lanes == lane),
                            rows_i[r], acc)
        gi = jnp.sum(acc)
        top_s = jnp.where(lanes == k, gm, top_s)
        top_i = jnp.where(lanes == k, gi, top_i)
    return top_s, top_i


def _local_top5(score_ref, base):
    lanes = _lane_iota()

    def body(j, carry):
        ts0, ts1, ts2, ts3, ts4, ti0, ti1, ti2, ti3, ti4 = carry
        cv = score_ref[pl.ds(j * LANES, LANES)]
        ci = (base + j * LANES) + lanes
        ts = [ts0, ts1, ts2, ts3, ts4]
        ti = [ti0, ti1, ti2, ti3, ti4]
        for k in range(TOP_K):
            pred = cv > ts[k]
            ns = jnp.where(pred, cv, ts[k])
            ni = jnp.where(pred, ci, ti[k])
            cv = jnp.where(pred, ts[k], cv)
            ci = jnp.where(pred, ti[k], ci)
            ts[k], ti[k] = ns, ni
        return (*ts, *ti)

    init = ([jnp.full((LANES,), NEG, jnp.float32)] * TOP_K
            + [jnp.zeros((LANES,), jnp.int32)] * TOP_K)
    carry = lax.fori_loop(0, NVREG, body, tuple(init))
    return _merge_rows(list(carry[:TOP_K]), list(carry[TOP_K:]))


def _sc_body(scores_hbm, sol_hbm,
             ts_out, ti_out, w_out, comb_out, misc_out, hbm_s, hbm_i,
             local_v, cand_s_v, cand_i_v, stage_s, stage_i, idx_v,
             rows_v, comb_v, sem):
    cid = lax.axis_index("c")
    sid = lax.axis_index("s")
    base = sid * PER_T
    lanes = _lane_iota()

    @pl.when(cid == 0)
    def _():
        pltpu.sync_copy(scores_hbm.at[pl.ds(base, PER_T)], local_v)

        loc_s, loc_i = _local_top5(local_v, base)

        stage_s[...] = loc_s
        stage_i[...] = loc_i
        pltpu.sync_copy(stage_s, hbm_s.at[sid])
        pltpu.sync_copy(stage_i, hbm_i.at[sid])

    plsc.subcore_barrier()

    @pl.when(jnp.logical_and(cid == 0, sid == 0))
    def _():
        pltpu.sync_copy(hbm_s, cand_s_v)
        pltpu.sync_copy(hbm_i, cand_i_v)

        top_s, top_i = _merge_rows([cand_s_v[r] for r in range(NT)],
                                   [cand_i_v[r] for r in range(NT)])

        valid = lanes < TOP_K
        s0 = jnp.max(top_s)
        logits = (top_s - s0) * INV_SQRT_SD
        w_un = jnp.where(valid, jnp.exp(logits), 0.0)
        weights = w_un / jnp.sum(w_un)

        idx_v[...] = jnp.where(valid, top_i, 0)
        pltpu.async_copy(sol_hbm.at[idx_v], rows_v, sem).wait()

        for c in range(SD // LANES):
            sl = pl.ds(c * LANES, LANES)
            acc = jnp.zeros((LANES,), jnp.float32)
            for k in range(TOP_K):
                wk = _extract_at_lane(weights, k)
                acc = acc + wk * rows_v[k, sl]
            comb_v[sl] = acc

        conf = s0
        n_used = jnp.sum(jnp.where(
            jnp.logical_and(valid, top_s > THRESH), 1.0, 0.0))
        misc = jnp.where(lanes == 0, conf, 0.0)
        misc = jnp.where(lanes == 1, n_used, misc)

        stage_s[...] = jnp.where(valid, top_s, 0.0)
        pltpu.sync_copy(stage_s, ts_out)
        stage_i[...] = top_i
        pltpu.sync_copy(stage_i, ti_out)
        stage_s[...] = weights
        pltpu.sync_copy(stage_s, w_out)
        pltpu.sync_copy(comb_v, comb_out)
        stage_s[...] = misc
        pltpu.sync_copy(stage_s, misc_out)


def _sc_topk(scores_pad, solution_memory):
    mesh = plsc.VectorSubcoreMesh(
        core_axis_name="c", subcore_axis_name="s",
        num_cores=NC, num_subcores=NS)
    fn = pl.kernel(
        _sc_body,
        out_type=(
            jax.ShapeDtypeStruct((LANES,), jnp.float32),
            jax.ShapeDtypeStruct((LANES,), jnp.int32),
            jax.ShapeDtypeStruct((LANES,), jnp.float32),
            jax.ShapeDtypeStruct((SD,), jnp.float32),
            jax.ShapeDtypeStruct((LANES,), jnp.float32),
            jax.ShapeDtypeStruct((NT, LANES), jnp.float32),
            jax.ShapeDtypeStruct((NT, LANES), jnp.int32),
        ),
        mesh=mesh,
        scratch_types=[
            pltpu.VMEM((PER_T,), jnp.float32),
            pltpu.VMEM((NT, LANES), jnp.float32),
            pltpu.VMEM((NT, LANES), jnp.int32),
            pltpu.VMEM((LANES,), jnp.float32),
            pltpu.VMEM((LANES,), jnp.int32),
            pltpu.VMEM((LANES,), jnp.int32),
            pltpu.VMEM((LANES, SD), jnp.float32),
            pltpu.VMEM((SD,), jnp.float32),
            pltpu.SemaphoreType.DMA,
        ],
        compiler_params=pltpu.CompilerParams(needs_layout_passes=False),
    )
    return fn(scores_pad, solution_memory)


def _output_body(x_ref, comb_ref, wo_ref, bo_ref, out_ref):
    e = jnp.dot(comb_ref[0, :], wo_ref[...],
                preferred_element_type=jnp.float32,
                precision=lax.Precision.HIGHEST) + bo_ref[0, :]
    g = jnp.dot(x_ref[...], e, preferred_element_type=jnp.float32,
                precision=lax.Precision.HIGHEST)
    gate = jax.nn.sigmoid(g)[:, None]
    out_ref[...] = gate * e[None, :] + (1.0 - gate) * x_ref[...]


def _blend_output(x2d, comb2d, Wo, bo2d):
    return pl.pallas_call(
        _output_body,
        out_shape=jax.ShapeDtypeStruct((S, H), jnp.float32),
    )(x2d, comb2d, Wo, bo2d)


@jax.jit
def kernel(x, problem_memory, solution_memory, confidence_memory,
           Wp, bp, Wo, bo, pattern_usage, pattern_success):
    B, S_, H_ = x.shape
    x2d = x.reshape(S_, H_)

    query = _project_query(x2d, Wp, bp.reshape(1, PD))
    scores = _compute_scores(query, problem_memory, confidence_memory,
                             pattern_usage, pattern_success)
    scores_pad = jnp.concatenate(
        [scores, jnp.full((M_PAD - M,), NEG, jnp.float32)])

    ts, ti, w, comb, misc, _, _ = _sc_topk(scores_pad, solution_memory)

    out2d = _blend_output(x2d, comb.reshape(1, SD), Wo, bo.reshape(1, H))

    output = out2d.reshape(B, S_, H_)
    top_scores = ts[:TOP_K][None, :]
    top_indices = ti[:TOP_K][None, :]
    attention_weights = w[:TOP_K][None, :]
    confidence = misc[0:1]
    num_patterns_used = misc[1:2].astype(jnp.int32)
    return (output, top_indices, top_scores, attention_weights,
            confidence, num_patterns_used)


# speedup vs baseline: 1.3271x; 1.0030x over previous
import math

import jax
import jax.numpy as jnp
from jax import lax
from jax.experimental import pallas as pl
from jax.experimental.pallas import tpu as pltpu
from jax.experimental.pallas import tpu_sc as plsc

M = 500000
PD = 128
SD = 128
H = 768
S = 2048
TOP_K = 5
THRESH = 0.5

NC = 2
NS = 16
NW = NC * NS
LANES = 16

NT = NS
PER_T = ((M + NT * LANES - 1) // (NT * LANES)) * LANES
M_PAD = PER_T * NT
NVREG = PER_T // LANES

NEG = -1e30
INV_SQRT_SD = 1.0 / math.sqrt(SD)


def _query_body(x_ref, wp_ref, bp_ref, out_ref):
    pc = jnp.mean(x_ref[...], axis=0)[None, :]
    q = jnp.dot(pc, wp_ref[...], preferred_element_type=jnp.float32)
    out_ref[...] = q + bp_ref[...]


def _project_query(x2d, Wp, bp2d):
    return pl.pallas_call(
        _query_body,
        out_shape=jax.ShapeDtypeStruct((1, PD), jnp.float32),
    )(x2d, Wp, bp2d)


_CHUNK = 25000
_NCHUNK = M // _CHUNK


def _scores_body(q_ref, pm_ref, conf_ref, use_ref, succ_ref, out_ref):
    sim = lax.dot_general(q_ref[...], pm_ref[...],
                          dimension_numbers=(((1,), (1,)), ((), ())),
                          preferred_element_type=jnp.float32)[0]
    use = use_ref[0, 0, :]
    out_ref[0, 0, :] = ((sim + 0.1 * jnp.log(use + 1.0))
                        + 0.2 * conf_ref[0, 0, :]
                        + 0.3 * (succ_ref[0, 0, :] / (use + 1e-8)))


def _compute_scores(query, problem_memory, confidence_memory, usage, success):
    vec3 = lambda v: v.reshape(_NCHUNK, 1, _CHUNK)
    out = pl.pallas_call(
        _scores_body,
        grid=(_NCHUNK,),
        in_specs=[
            pl.BlockSpec((1, PD), lambda i: (0, 0)),
            pl.BlockSpec((_CHUNK, PD), lambda i: (i, 0)),
            pl.BlockSpec((1, 1, _CHUNK), lambda i: (i, 0, 0)),
            pl.BlockSpec((1, 1, _CHUNK), lambda i: (i, 0, 0)),
            pl.BlockSpec((1, 1, _CHUNK), lambda i: (i, 0, 0)),
        ],
        out_specs=pl.BlockSpec((1, 1, _CHUNK), lambda i: (i, 0, 0)),
        out_shape=jax.ShapeDtypeStruct((_NCHUNK, 1, _CHUNK), jnp.float32),
    )(query, problem_memory, vec3(confidence_memory[:, 0]), vec3(usage),
      vec3(success))
    return out.reshape(M)


def _lane_iota():
    return lax.iota(jnp.int32, LANES)


def _extract_at_lane(vec, lane):
    return jnp.sum(jnp.where(_lane_iota() == lane, vec, 0))


def _merge_rows(rows_s, rows_i):
    lanes = _lane_iota()
    nrows = len(rows_s)
    top_s = jnp.full((LANES,), NEG, jnp.float32)
    top_i = jnp.zeros((LANES,), jnp.int32)
    chosen = []
    for k in range(TOP_K):
        m = jnp.full((LANES,), NEG, jnp.float32)
        arow = jnp.zeros((LANES,), jnp.int32)
        for r in range(nrows):
            v = rows_s[r]
            flatpos = r * LANES + lanes
            for c in chosen:
                v = jnp.where(flatpos == c, NEG, v)
            pred = v > m
            m = jnp.where(pred, v, m)
            arow = jnp.where(pred, r, arow)
        gm = jnp.max(m)
        pred = (m == gm)
        first = jnp.logical_and(pred, jnp.cumsum(pred.astype(jnp.int32)) == 1)
        lane = jnp.sum(jnp.where(first, lanes, 0))
        rbest = jnp.sum(jnp.where(first, arow, 0))
        chosen.append(rbest * LANES + lane)
        acc = jnp.zeros((LANES,), jnp.int32)
        for r in range(nrows):
            acc = jnp.where(jnp.logical_and(rbest == r, lanes == lane),
                            rows_i[r], acc)
        gi = jnp.sum(acc)
        top_s = jnp.where(lanes == k, gm, top_s)
        top_i = jnp.where(lanes == k, gi, top_i)
    return top_s, top_i


def _local_top5(score_ref, base):
    lanes = _lane_iota()

    def body(j, carry):
        ts0, ts1, ts2, ts3, ts4, ti0, ti1, ti2, ti3, ti4 = carry
        cv = score_ref[pl.ds(j * LANES, LANES)]
        ci = (base + j * LANES) + lanes
        ts = [ts0, ts1, ts2, ts3, ts4]
        ti = [ti0, ti1, ti2, ti3, ti4]
        for k in range(TOP_K):
            pred = cv > ts[k]
            ns = jnp.where(pred, cv, ts[k])
            ni = jnp.where(pred, ci, ti[k])
            cv = jnp.where(pred, ts[k], cv)
            ci = jnp.where(pred, ti[k], ci)
            ts[k], ti[k] = ns, ni
        return (*ts, *ti)

    init = ([jnp.full((LANES,), NEG, jnp.float32)] * TOP_K
            + [jnp.zeros((LANES,), jnp.int32)] * TOP_K)
    carry = lax.fori_loop(0, NVREG, body, tuple(init))
    return _merge_rows(list(carry[:TOP_K]), list(carry[TOP_K:]))


def _sc_body(scores_hbm, sol_hbm,
             ts_out, ti_out, w_out, comb_out, misc_out, hbm_s, hbm_i,
             local_v, cand_s_v, cand_i_v, stage_s, stage_i, idx_v,
             rows_v, comb_v, sem):
    cid = lax.axis_index("c")
    sid = lax.axis_index("s")
    base = sid * PER_T
    lanes = _lane_iota()

    @pl.when(cid == 0)
    def _():
        pltpu.sync_copy(scores_hbm.at[pl.ds(base, PER_T)], local_v)

        loc_s, loc_i = _local_top5(local_v, base)

        stage_s[...] = loc_s
        stage_i[...] = loc_i
        pltpu.sync_copy(stage_s, hbm_s.at[sid])
        pltpu.sync_copy(stage_i, hbm_i.at[sid])

    plsc.subcore_barrier()

    @pl.when(jnp.logical_and(cid == 0, sid == 0))
    def _():
        pltpu.sync_copy(hbm_s, cand_s_v)
        pltpu.sync_copy(hbm_i, cand_i_v)

        top_s, top_i = _merge_rows([cand_s_v[r] for r in range(NT)],
                                   [cand_i_v[r] for r in range(NT)])

        valid = lanes < TOP_K
        s0 = jnp.max(top_s)
        logits = (top_s - s0) * INV_SQRT_SD
        w_un = jnp.where(valid, jnp.exp(logits), 0.0)
        weights = w_un / jnp.sum(w_un)

        idx_v[...] = jnp.where(valid, top_i, 0)
        pltpu.async_copy(sol_hbm.at[idx_v], rows_v, sem).wait()

        for c in range(SD // LANES):
            sl = pl.ds(c * LANES, LANES)
            acc = jnp.zeros((LANES,), jnp.float32)
            for k in range(TOP_K):
                wk = _extract_at_lane(weights, k)
                acc = acc + wk * rows_v[k, sl]
            comb_v[sl] = acc

        conf = s0
        n_used = jnp.sum(jnp.where(
            jnp.logical_and(valid, top_s > THRESH), 1.0, 0.0))
        misc = jnp.where(lanes == 0, conf, 0.0)
        misc = jnp.where(lanes == 1, n_used, misc)

        stage_s[...] = jnp.where(valid, top_s, 0.0)
        pltpu.sync_copy(stage_s, ts_out)
        stage_i[...] = top_i
        pltpu.sync_copy(stage_i, ti_out)
        stage_s[...] = weights
        pltpu.sync_copy(stage_s, w_out)
        pltpu.sync_copy(comb_v, comb_out)
        stage_s[...] = misc
        pltpu.sync_copy(stage_s, misc_out)


def _sc_topk(scores_pad, solution_memory):
    mesh = plsc.VectorSubcoreMesh(
        core_axis_name="c", subcore_axis_name="s",
        num_cores=NC, num_subcores=NS)
    fn = pl.kernel(
        _sc_body,
        out_type=(
            jax.ShapeDtypeStruct((LANES,), jnp.float32),
            jax.ShapeDtypeStruct((LANES,), jnp.int32),
            jax.ShapeDtypeStruct((LANES,), jnp.float32),
            jax.ShapeDtypeStruct((SD,), jnp.float32),
            jax.ShapeDtypeStruct((LANES,), jnp.float32),
            jax.ShapeDtypeStruct((NT, LANES), jnp.float32),
            jax.ShapeDtypeStruct((NT, LANES), jnp.int32),
        ),
        mesh=mesh,
        scratch_types=[
            pltpu.VMEM((PER_T,), jnp.float32),
            pltpu.VMEM((NT, LANES), jnp.float32),
            pltpu.VMEM((NT, LANES), jnp.int32),
            pltpu.VMEM((LANES,), jnp.float32),
            pltpu.VMEM((LANES,), jnp.int32),
            pltpu.VMEM((LANES,), jnp.int32),
            pltpu.VMEM((LANES, SD), jnp.float32),
            pltpu.VMEM((SD,), jnp.float32),
            pltpu.SemaphoreType.DMA,
        ],
        compiler_params=pltpu.CompilerParams(needs_layout_passes=False),
    )
    return fn(scores_pad, solution_memory)


def _output_body(x_ref, comb_ref, wo_ref, bo_ref, out_ref):
    e = jnp.dot(comb_ref[0, :], wo_ref[...],
                preferred_element_type=jnp.float32,
                precision=lax.Precision.HIGHEST) + bo_ref[0, :]
    g = jnp.dot(x_ref[...], e, preferred_element_type=jnp.float32,
                precision=lax.Precision.HIGHEST)
    gate = jax.nn.sigmoid(g)[:, None]
    out_ref[...] = gate * e[None, :] + (1.0 - gate) * x_ref[...]


def _blend_output(x2d, comb2d, Wo, bo2d):
    return pl.pallas_call(
        _output_body,
        out_shape=jax.ShapeDtypeStruct((S, H), jnp.float32),
    )(x2d, comb2d, Wo, bo2d)


@jax.jit
def kernel(x, problem_memory, solution_memory, confidence_memory,
           Wp, bp, Wo, bo, pattern_usage, pattern_success):
    B, S_, H_ = x.shape
    x2d = x.reshape(S_, H_)

    query = _project_query(x2d, Wp, bp.reshape(1, PD))
    scores = _compute_scores(query, problem_memory, confidence_memory,
                             pattern_usage, pattern_success)
    scores_pad = jnp.concatenate(
        [scores, jnp.full((M_PAD - M,), NEG, jnp.float32)])

    ts, ti, w, comb, misc, _, _ = _sc_topk(scores_pad, solution_memory)

    out2d = _blend_output(x2d, comb.reshape(1, SD), Wo, bo.reshape(1, H))

    output = out2d.reshape(B, S_, H_)
    top_scores = ts[:TOP_K][None, :]
    top_indices = ti[:TOP_K][None, :]
    attention_weights = w[:TOP_K][None, :]
    confidence = misc[0:1]
    num_patterns_used = misc[1:2].astype(jnp.int32)
    return (output, top_indices, top_scores, attention_weights,
            confidence, num_patterns_used)
